# baseline (speedup 1.0000x reference)
import functools

import jax
import jax.numpy as jnp
from jax.experimental import pallas as pl
from jax.experimental.pallas import tpu as pltpu

_GO = -11.0
_GE = -1.0
_NEG = -1e30
_A = 1023
_N = 2 * _A - 1
_KP = 1024
_U = 8
_NPAD = 2048


def _skew(mat):
    a, b = mat.shape
    n = a + b - 1
    p = jnp.pad(mat, ((0, 0), (0, n + 1 - b)), constant_values=_NEG)
    return p.reshape(-1)[: a * n].reshape(a, n)


def _shift_k(u, fill):
    return jnp.concatenate(
        [jnp.full(u.shape[:-1] + (1,), fill, u.dtype), u[..., :-1]], axis=-1)


_CGO = 1.670170079024566e-05
_CGE = 0.36787944117144233

_C = 4
_CW = _KP // _C


def _sw_kernel(y_ref, z_ref, out_ref,
               h1_ref, h2_ref, h3_ref, r1_ref, f1_ref, f2_ref,
               m_ref, s_ref):
    t = pl.program_id(0)

    @pl.when(t == 0)
    def _():
        h1_ref[...] = jnp.zeros((1, _KP), jnp.float32)
        h2_ref[...] = jnp.zeros((1, _KP), jnp.float32)
        h3_ref[...] = jnp.zeros((1, _KP), jnp.float32)
        r1_ref[...] = jnp.full((1, _KP), _NEG, jnp.float32)
        f1_ref[...] = jnp.full((1, _KP), _NEG, jnp.float32)
        f2_ref[...] = jnp.full((1, _KP), _NEG, jnp.float32)
        m_ref[...] = jnp.full((1, _KP), _NEG, jnp.float32)
        s_ref[...] = jnp.zeros((1, _KP), jnp.float32)

    sls = [slice(c * _CW, (c + 1) * _CW) for c in range(_C)]
    H1 = [h1_ref[0, s] for s in sls]
    H2 = [h2_ref[0, s] for s in sls]
    H3 = [h3_ref[0, s] for s in sls]
    R1 = [r1_ref[0, s] for s in sls]
    F1 = [f1_ref[0, s] for s in sls]
    F2 = [f2_ref[0, s] for s in sls]
    Mm = [m_ref[0, s] for s in sls]
    Ss = [s_ref[0, s] for s in sls]

    zero1 = jnp.zeros((1,), jnp.float32)
    neg1 = jnp.full((1,), _NEG, jnp.float32)

    for u in range(_U):
        y = y_ref[u, 0, :]
        z = z_ref[u, 0, :]
        nH1, nR1, nF1 = [], [], []
        for c in range(_C):
            y_c = y[sls[c]]
            z_c = z[sls[c]]
            hb = zero1 if c == 0 else H3[c - 1][_CW - 1:]
            fb = neg1 if c == 0 else F2[c - 1][_CW - 1:]

            aln = y_c + jnp.concatenate([hb, H2[c][:_CW - 1]])
            rgt = R1[c]
            dwn = jnp.concatenate([fb, F1[c][:_CW - 1]])

            q = jnp.maximum(jnp.maximum(aln, rgt), jnp.maximum(dwn, 0.0))
            ea = jnp.exp(aln - q)
            er = jnp.exp(rgt - q)
            ed = jnp.exp(dwn - q)
            e0 = jnp.exp(-q)
            se3 = ea + er + ed

            nH1.append(q + jnp.log(se3 + e0))
            nR1.append(q + jnp.log(ea * _CGO + er * _CGE))
            nF1.append(q + jnp.log((ea + er) * _CGO + ed * _CGE))

            qz = q + z_c
            keep = Mm[c] >= qz
            e = jnp.exp(jnp.where(keep, qz - Mm[c], Mm[c] - qz))
            Ss[c] = jnp.where(keep, Ss[c] + se3 * e, Ss[c] * e + se3)
            Mm[c] = jnp.maximum(Mm[c], qz)

        H3 = H2
        H2 = H1
        H1 = nH1
        F2 = F1
        F1 = nF1
        R1 = nR1

    for c in range(_C):
        h1_ref[0, sls[c]] = H1[c]
        h2_ref[0, sls[c]] = H2[c]
        h3_ref[0, sls[c]] = H3[c]
        r1_ref[0, sls[c]] = R1[c]
        f1_ref[0, sls[c]] = F1[c]
        f2_ref[0, sls[c]] = F2[c]
        m_ref[0, sls[c]] = Mm[c]
        s_ref[0, sls[c]] = Ss[c]

    @pl.when(t == _NPAD // _U - 1)
    def _():
        mm = m_ref[...]
        gm = jnp.max(mm, keepdims=True)
        ssum = jnp.sum(s_ref[...] * jnp.exp(mm - gm), keepdims=True)
        out_ref[...] = (gm + jnp.log(ssum)).reshape(1, 1)


@jax.jit
def _sw_loss(x):
    xc = x[:-1, :-1]
    x2 = x[1:, 1:]
    def _prep(mat):
        s = jnp.pad(_skew(mat).T, ((0, _NPAD - _N), (0, _KP - _A)),
                    constant_values=_NEG)
        cols = [
            jnp.pad(s[:, c * _CW:(c + 1) * _CW], ((c, 0), (0, 0)),
                    constant_values=_NEG)[:_NPAD]
            for c in range(_C)
        ]
        return jnp.concatenate(cols, axis=1).reshape(_NPAD, 1, _KP)

    yt = _prep(xc)
    zt = _prep(x2)

    out = pl.pallas_call(
        _sw_kernel,
        grid=(_NPAD // _U,),
        in_specs=[
            pl.BlockSpec((_U, 1, _KP), lambda i: (i, 0, 0)),
            pl.BlockSpec((_U, 1, _KP), lambda i: (i, 0, 0)),
        ],
        out_specs=pl.BlockSpec((1, 1), lambda i: (0, 0)),
        out_shape=jax.ShapeDtypeStruct((1, 1), jnp.float32),
        scratch_shapes=[pltpu.VMEM((1, _KP), jnp.float32)
                        for _ in range(8)],
        compiler_params=pltpu.CompilerParams(
            dimension_semantics=("arbitrary",)),
    )(yt, zt)
    return out[0, 0]


def kernel(x):
    return _sw_loss(x)


# speedup vs baseline: 1.0794x; 1.0794x over previous
import functools

import jax
import jax.numpy as jnp
from jax.experimental import pallas as pl
from jax.experimental.pallas import tpu as pltpu

_GO = -11.0
_GE = -1.0
_NEG = -1e30
_A = 1023
_N = 2 * _A - 1
_KP = 1024
_U = 16
_NPAD = 2048


def _skew(mat):
    a, b = mat.shape
    n = a + b - 1
    p = jnp.pad(mat, ((0, 0), (0, n + 1 - b)), constant_values=_NEG)
    return p.reshape(-1)[: a * n].reshape(a, n)


def _shift_k(u, fill):
    return jnp.concatenate(
        [jnp.full(u.shape[:-1] + (1,), fill, u.dtype), u[..., :-1]], axis=-1)


_CGO = 1.670170079024566e-05
_CGE = 0.36787944117144233


def _sw_kernel(y_ref, z_ref, out_ref,
               h2_ref, h1_ref, r1_ref, f1_ref, m_ref, s_ref):
    t = pl.program_id(0)

    @pl.when(t == 0)
    def _():
        h2_ref[...] = jnp.zeros((1, _KP), jnp.float32)
        h1_ref[...] = jnp.zeros((1, _KP), jnp.float32)
        r1_ref[...] = jnp.full((1, _KP), _NEG, jnp.float32)
        f1_ref[...] = jnp.full((1, _KP), _NEG, jnp.float32)
        m_ref[...] = jnp.full((1, _KP), _NEG, jnp.float32)
        s_ref[...] = jnp.zeros((1, _KP), jnp.float32)

    h2 = h2_ref[0, :]
    h1 = h1_ref[0, :]
    r1 = r1_ref[0, :]
    f1 = f1_ref[0, :]
    m_run = m_ref[0, :]
    s_run = s_ref[0, :]

    for u in range(_U):
        y = y_ref[u, 0, :]
        z = z_ref[u, 0, :]

        aln = y + _shift_k(h2, 0.0)
        rgt = r1
        dwn = _shift_k(f1, _NEG)

        q = jnp.maximum(jnp.maximum(aln, rgt), jnp.maximum(dwn, 0.0))
        ea = jnp.exp(aln - q)
        er = jnp.exp(rgt - q)
        ed = jnp.exp(dwn - q)
        e0 = jnp.exp(-q)
        se3 = ea + er + ed

        h0 = q + jnp.log(se3 + e0)
        r0 = q + jnp.log(ea * _CGO + er * _CGE)
        f0 = q + jnp.log((ea + er) * _CGO + ed * _CGE)

        qz = q + z
        keep = m_run >= qz
        e = jnp.exp(jnp.where(keep, qz - m_run, m_run - qz))
        s_run = jnp.where(keep, s_run + se3 * e, s_run * e + se3)
        m_run = jnp.maximum(m_run, qz)

        h2 = h1
        h1 = h0
        r1 = r0
        f1 = f0

    h2_ref[0, :] = h2
    h1_ref[0, :] = h1
    r1_ref[0, :] = r1
    f1_ref[0, :] = f1
    m_ref[0, :] = m_run
    s_ref[0, :] = s_run

    @pl.when(t == _NPAD // _U - 1)
    def _():
        mm = m_ref[...]
        gm = jnp.max(mm, keepdims=True)
        ssum = jnp.sum(s_ref[...] * jnp.exp(mm - gm), keepdims=True)
        out_ref[...] = (gm + jnp.log(ssum)).reshape(1, 1)


@jax.jit
def _sw_loss(x):
    xc = x[:-1, :-1]
    x2 = x[1:, 1:]
    yt = jnp.pad(_skew(xc).T, ((0, _NPAD - _N), (0, _KP - _A)),
                 constant_values=_NEG).reshape(_NPAD, 1, _KP)
    zt = jnp.pad(_skew(x2).T, ((0, _NPAD - _N), (0, _KP - _A)),
                 constant_values=_NEG).reshape(_NPAD, 1, _KP)

    out = pl.pallas_call(
        _sw_kernel,
        grid=(_NPAD // _U,),
        in_specs=[
            pl.BlockSpec((_U, 1, _KP), lambda i: (i, 0, 0)),
            pl.BlockSpec((_U, 1, _KP), lambda i: (i, 0, 0)),
        ],
        out_specs=pl.BlockSpec((1, 1), lambda i: (0, 0)),
        out_shape=jax.ShapeDtypeStruct((1, 1), jnp.float32),
        scratch_shapes=[
            pltpu.VMEM((1, _KP), jnp.float32),
            pltpu.VMEM((1, _KP), jnp.float32),
            pltpu.VMEM((1, _KP), jnp.float32),
            pltpu.VMEM((1, _KP), jnp.float32),
            pltpu.VMEM((1, _KP), jnp.float32),
            pltpu.VMEM((1, _KP), jnp.float32),
        ],
        compiler_params=pltpu.CompilerParams(
            dimension_semantics=("arbitrary",)),
    )(yt, zt)
    return out[0, 0]


def kernel(x):
    return _sw_loss(x)
